# baseline (speedup 1.0000x reference)
import functools

import jax
import jax.numpy as jnp
from jax import lax
from jax.experimental import pallas as pl
from jax.experimental.pallas import tpu as pltpu
from jax.experimental.pallas import tpu_sc as plsc

N_NODES = 10000
N_EDGES = 320000
D_FEAT = 128
NUM_CORES = 2
NUM_SUBCORES = 16
NUM_WORKERS = NUM_CORES * NUM_SUBCORES
EDGES_PER_WORKER = N_EDGES // NUM_WORKERS
CHUNK = 80
NUM_CHUNKS = EDGES_PER_WORKER // CHUNK
GROUPS = CHUNK // 16

_BITREV = [int(f"{j:04b}"[::-1], 2) for j in range(16)]


def _decoder_body(s_hbm, t_hbm, si_hbm, di_hbm, out_hbm,
                  sidx, didx, srows, trows, outv,
                  sem_s0, sem_s1, sem_t0, sem_t1):
    wid = lax.axis_index("s") * NUM_CORES + lax.axis_index("c")
    base = wid * EDGES_PER_WORKER
    pltpu.sync_copy(si_hbm.at[pl.ds(base, EDGES_PER_WORKER)], sidx)
    pltpu.sync_copy(di_hbm.at[pl.ds(base, EDGES_PER_WORKER)], didx)
    lanes = lax.iota(jnp.int32, 16)
    bfly_idx = {k: lanes ^ k for k in (8, 4, 2, 1)}
    seg_mask = {k: (lanes & k) == 0 for k in (8, 4, 2, 1)}
    sem_s = (sem_s0, sem_s1)
    sem_t = (sem_t0, sem_t1)

    def gather_start(ci, b):
        off = pl.multiple_of(ci * CHUNK, 8)
        pltpu.async_copy(s_hbm.at[sidx.at[pl.ds(off, CHUNK)]], srows.at[b], sem_s[b])
        pltpu.async_copy(t_hbm.at[didx.at[pl.ds(off, CHUNK)]], trows.at[b], sem_t[b])

    def gather_wait(b):
        pltpu.make_async_copy(s_hbm.at[pl.ds(0, CHUNK)], srows.at[b], sem_s[b]).wait()
        pltpu.make_async_copy(t_hbm.at[pl.ds(0, CHUNK)], trows.at[b], sem_t[b]).wait()

    def edge_dot(b, e):
        acc = srows[b, e, pl.ds(0, 16)] * trows[b, e, pl.ds(0, 16)]
        for k in range(1, 8):
            acc = acc + srows[b, e, pl.ds(k * 16, 16)] * trows[b, e, pl.ds(k * 16, 16)]
        return acc

    def bfly_add(v, k):
        return v + v.at[bfly_idx[k]].get(mode="promise_in_bounds")

    def compute(ci, b):
        off = ci * CHUNK

        @plsc.parallel_loop(0, GROUPS)
        def group_body(gi):
            e0 = gi * 16
            vs = [edge_dot(b, e0 + _BITREV[j]) for j in range(16)]
            for k in (8, 4, 2, 1):
                vs = [jnp.where(seg_mask[k], bfly_add(a, k), bfly_add(c, k))
                      for a, c in zip(vs[0::2], vs[1::2])]
            outv[pl.ds(off + e0, 16)] = vs[0]

    gather_start(0, 0)
    gather_start(1, 1)

    def pair_body(p, carry):
        ci0 = 2 * p
        for b in range(2):
            ci = ci0 + b
            gather_wait(b)
            compute(ci, b)

            @pl.when(ci + 2 < NUM_CHUNKS)
            def _():
                gather_start(ci + 2, b)
        return carry

    lax.fori_loop(0, NUM_CHUNKS // 2, pair_body, 0)
    gather_wait(0)
    compute(NUM_CHUNKS - 1, 0)
    pltpu.sync_copy(outv, out_hbm.at[pl.ds(base, EDGES_PER_WORKER)])


@functools.partial(jax.jit)
def kernel(s, t, edge_index):
    ei = edge_index.astype(jnp.int32)
    mesh = plsc.VectorSubcoreMesh(core_axis_name="c", subcore_axis_name="s")
    run = pl.kernel(
        _decoder_body,
        out_type=jax.ShapeDtypeStruct((N_EDGES,), jnp.float32),
        mesh=mesh,
        scratch_types=[
            pltpu.VMEM((EDGES_PER_WORKER,), jnp.int32),
            pltpu.VMEM((EDGES_PER_WORKER,), jnp.int32),
            pltpu.VMEM((2, CHUNK, D_FEAT), jnp.float32),
            pltpu.VMEM((2, CHUNK, D_FEAT), jnp.float32),
            pltpu.VMEM((EDGES_PER_WORKER,), jnp.float32),
            pltpu.SemaphoreType.DMA,
            pltpu.SemaphoreType.DMA,
            pltpu.SemaphoreType.DMA,
            pltpu.SemaphoreType.DMA,
        ],
    )
    return run(s, t, ei[0], ei[1])
